# Initial kernel scaffold; baseline (speedup 1.0000x reference)
import jax
import jax.numpy as jnp
from jax.experimental import pallas as pl


def kernel(pc1, pc2, feat1, feat2, pos_w, pos_b, mlp_w1, mlp_b1, mlp_w2, mlp_b2, t1_w, t1_b, t2_w, t2_b):
    raise NotImplementedError("write your pallas kernel here")


# speedup vs baseline: 8.5330x; 8.5330x over previous
import functools

import jax
import jax.numpy as jnp
from jax import lax
from jax.experimental import pallas as pl

NSAMPLE = 16
LEAKY = 0.1
ROWS = 256


def _leaky(x):
    return jnp.where(x >= 0, x, LEAKY * x)


def _cross_body(q_ref, kT_ref, qf_ref, kf_ref, k_ref,
                pos_wT_ref, pos_b_ref, w1T_ref, b1_ref, w2T_ref, b2_ref,
                tT_ref, tb_ref, out_ref):
    q = q_ref[0]
    kT = kT_ref[0]
    qf = qf_ref[0]
    kf = kf_ref[0]
    kxyz = k_ref[0]
    pos_wT = pos_wT_ref[...]
    pos_b = pos_b_ref[...]
    w1T = w1T_ref[...]
    b1 = b1_ref[...]
    w2T = w2T_ref[...]
    b2 = b2_ref[...]

    R = q.shape[0]
    N = kT.shape[1]
    f32 = jnp.float32

    qk = jnp.dot(q, kT, preferred_element_type=f32)
    qn = jnp.sum(q * q, axis=1, keepdims=True)
    kn = jnp.sum(kT * kT, axis=0, keepdims=True)
    d = (-2.0 * qk + qn) + kn

    t_pre = kf + jnp.dot(kxyz, pos_wT, preferred_element_type=f32)
    base = qf + (pos_b - jnp.dot(q, pos_wT, preferred_element_type=f32))

    iota = lax.broadcasted_iota(jnp.int32, (R, N), 1)

    def body(_, carry):
        dist, mx = carry
        m = jnp.min(dist, axis=1, keepdims=True)
        eq = dist == m
        idx = jnp.min(jnp.where(eq, iota, N), axis=1, keepdims=True)
        oh = iota == idx
        dist = jnp.where(oh, jnp.float32(jnp.inf), dist)
        g = jnp.dot(oh.astype(f32), t_pre, preferred_element_type=f32)
        x = _leaky(g + base)
        x = _leaky(jnp.dot(x, w1T, preferred_element_type=f32) + b1)
        x = _leaky(jnp.dot(x, w2T, preferred_element_type=f32) + b2)
        return dist, jnp.maximum(mx, x)

    mx0 = jnp.full((R, 32), -jnp.inf, dtype=f32)
    _, mx = lax.fori_loop(0, NSAMPLE, body, (d, mx0))

    out_ref[0] = jnp.dot(mx, tT_ref[...], preferred_element_type=f32) + tb_ref[...]


def _cross_dir(pcq, pck, featq, featk, pos_wT, pos_b2, w1T, b12, w2T, b22,
               tT, tb2, interpret=False):
    B, N, _ = pcq.shape
    D = featq.shape[-1]
    R = ROWS
    kT = jnp.swapaxes(pck, 1, 2)
    grid = (B, N // R)

    specs = [
        pl.BlockSpec((1, R, 3), lambda b, i: (b, i, 0)),
        pl.BlockSpec((1, 3, N), lambda b, i: (b, 0, 0)),
        pl.BlockSpec((1, R, D), lambda b, i: (b, i, 0)),
        pl.BlockSpec((1, N, D), lambda b, i: (b, 0, 0)),
        pl.BlockSpec((1, N, 3), lambda b, i: (b, 0, 0)),
        pl.BlockSpec((3, D), lambda b, i: (0, 0)),
        pl.BlockSpec((1, D), lambda b, i: (0, 0)),
        pl.BlockSpec((D, D), lambda b, i: (0, 0)),
        pl.BlockSpec((1, D), lambda b, i: (0, 0)),
        pl.BlockSpec((D, D), lambda b, i: (0, 0)),
        pl.BlockSpec((1, D), lambda b, i: (0, 0)),
        pl.BlockSpec((D, D), lambda b, i: (0, 0)),
        pl.BlockSpec((1, D), lambda b, i: (0, 0)),
    ]
    out_spec = pl.BlockSpec((1, R, D), lambda b, i: (b, i, 0))

    return pl.pallas_call(
        _cross_body,
        grid=grid,
        in_specs=specs,
        out_specs=out_spec,
        out_shape=jax.ShapeDtypeStruct((B, N, D), jnp.float32),
        interpret=interpret,
    )(pcq, kT, featq, featk, pck, pos_wT, pos_b2, w1T, b12, w2T, b22, tT, tb2)


@jax.jit
def kernel(pc1, pc2, feat1, feat2, pos_w, pos_b, mlp_w1, mlp_b1,
           mlp_w2, mlp_b2, t1_w, t1_b, t2_w, t2_b):
    pos_wT = pos_w.T
    pos_b2 = pos_b.reshape(1, -1)
    w1T = mlp_w1.T
    b12 = mlp_b1.reshape(1, -1)
    w2T = mlp_w2.T
    b22 = mlp_b2.reshape(1, -1)

    f1 = _cross_dir(pc1, pc2, feat1, feat2, pos_wT, pos_b2, w1T, b12,
                    w2T, b22, t1_w.T, t1_b.reshape(1, -1))
    f2 = _cross_dir(pc2, pc1, feat2, feat1, pos_wT, pos_b2, w1T, b12,
                    w2T, b22, t2_w.T, t2_b.reshape(1, -1))
    return (f1, f2)


# speedup vs baseline: 9.6109x; 1.1263x over previous
import functools

import jax
import jax.numpy as jnp
from jax import lax
from jax.experimental import pallas as pl
from jax.experimental.pallas import tpu as pltpu
from jax.experimental.pallas import tpu_sc as plsc

NSAMPLE = 16
LEAKY = 0.1
ROWS = 256
MLP_ROWS = 512

SC_CORES = 2
SC_SUBCORES = 16
SC_WORKERS = SC_CORES * SC_SUBCORES
GATHER_CHUNK = 128


def _leaky(x):
    return jnp.where(x >= 0, x, LEAKY * x)


def _topk_body(q_ref, kT_ref, kf_ref, kxyz_ref, pos_wT_ref,
               idx_ref, tpre_ref):
    b = pl.program_id(0)
    q = q_ref[0]
    kT = kT_ref[0]
    f32 = jnp.float32

    R = q.shape[0]
    N = kT.shape[1]

    tpre_ref[0] = kf_ref[0] + jnp.dot(kxyz_ref[0], pos_wT_ref[...],
                                      preferred_element_type=f32)

    qk = jnp.dot(q, kT, preferred_element_type=f32)
    qn = jnp.sum(q * q, axis=1, keepdims=True)
    kn = jnp.sum(kT * kT, axis=0, keepdims=True)
    d = (-2.0 * qk + qn) + kn

    iota = lax.broadcasted_iota(jnp.int32, (R, N), 1)
    lane16 = lax.broadcasted_iota(jnp.int32, (R, NSAMPLE), 1)
    gbase = b * N

    def body(k, carry):
        dist, idxbuf = carry
        m = jnp.min(dist, axis=1, keepdims=True)
        eq = dist == m
        idx = jnp.min(jnp.where(eq, iota, N), axis=1, keepdims=True)
        dist = jnp.where(iota == idx, jnp.float32(jnp.inf), dist)
        idxbuf = jnp.where(lane16 == k, idx + gbase, idxbuf)
        return dist, idxbuf

    idxbuf0 = jnp.zeros((R, NSAMPLE), dtype=jnp.int32)
    _, idxbuf = lax.fori_loop(0, NSAMPLE, body, (d, idxbuf0))
    idx_ref[0] = idxbuf


def _topk_call(pcq, pck, featk):
    B, N, _ = pcq.shape
    D = featk.shape[-1]
    R = ROWS
    grid = (B, N // R)

    specs = [
        pl.BlockSpec((1, R, 3), lambda b, i: (b, i, 0)),
        pl.BlockSpec((1, 3, N), lambda b, i: (b, 0, 0)),
        pl.BlockSpec((1, R, D), lambda b, i: (b, i, 0)),
        pl.BlockSpec((1, R, 3), lambda b, i: (b, i, 0)),
        pl.BlockSpec((3, D), lambda b, i: (0, 0)),
    ]
    out_specs = [
        pl.BlockSpec((1, R, NSAMPLE), lambda b, i: (b, i, 0)),
        pl.BlockSpec((1, R, D), lambda b, i: (b, i, 0)),
    ]
    out_shape = [
        jax.ShapeDtypeStruct((B, N, NSAMPLE), jnp.int32),
        jax.ShapeDtypeStruct((B, N, D), jnp.float32),
    ]
    return pl.pallas_call(
        _topk_body, grid=grid, in_specs=specs,
        out_specs=out_specs, out_shape=out_shape,
    )


def _run_topk(pcq, pck, featk, pos_wT):
    kT = jnp.swapaxes(pck, 1, 2)
    return _topk_call(pcq, pck, featk)(pcq, kT, featk, pck, pos_wT)


def _sc_gather(idx_flat, table):
    T = idx_flat.shape[0]
    D = table.shape[1]
    per_w = T // SC_WORKERS
    ch = GATHER_CHUNK
    n_ch = per_w // ch
    mesh = plsc.VectorSubcoreMesh(core_axis_name="c", subcore_axis_name="s",
                                  num_cores=SC_CORES, num_subcores=SC_SUBCORES)

    @functools.partial(
        pl.kernel, mesh=mesh,
        compiler_params=pltpu.CompilerParams(use_tc_tiling_on_sc=False),
        out_type=jax.ShapeDtypeStruct((T, D), jnp.float32),
        scratch_types=[
            pltpu.VMEM((ch,), jnp.int32),
            pltpu.VMEM((ch, D), jnp.float32),
            pltpu.SemaphoreType.DMA,
        ],
    )
    def gk(idx_hbm, table_hbm, out_hbm, idx_v, rows_v, sem):
        wid = lax.axis_index("s") * SC_CORES + lax.axis_index("c")
        base = wid * per_w

        def body(c, carry):
            off = base + c * ch
            pltpu.sync_copy(idx_hbm.at[pl.ds(off, ch)], idx_v)
            pltpu.async_copy(table_hbm.at[idx_v], rows_v, sem).wait()
            pltpu.sync_copy(rows_v, out_hbm.at[pl.ds(off, ch)])
            return carry

        lax.fori_loop(0, n_ch, body, 0)

    return gk(idx_flat, table)


def _mlp_body(g_ref, qf_ref, q_ref, pos_wT_ref, pos_b_ref,
              w1T_ref, b1_ref, w2T_ref, b2_ref, tT_ref, tb_ref, out_ref):
    f32 = jnp.float32
    g = g_ref[0]
    qf = qf_ref[0]
    q = q_ref[0]
    R = qf.shape[0]
    D = qf.shape[1]

    base = qf + (pos_b_ref[...]
                 - jnp.dot(q, pos_wT_ref[...], preferred_element_type=f32))
    x = _leaky(g + base[:, None, :])
    x2 = x.reshape(R * NSAMPLE, D)
    x2 = _leaky(jnp.dot(x2, w1T_ref[...], preferred_element_type=f32)
                + b1_ref[...])
    x2 = _leaky(jnp.dot(x2, w2T_ref[...], preferred_element_type=f32)
                + b2_ref[...])
    x3 = x2.reshape(R, NSAMPLE, D)
    mx = x3[:, 0, :]
    for k in range(1, NSAMPLE):
        mx = jnp.maximum(mx, x3[:, k, :])
    out_ref[0] = jnp.dot(mx, tT_ref[...], preferred_element_type=f32) + tb_ref[...]


def _mlp_call(g, featq, pcq, pos_wT, pos_b2, w1T, b12, w2T, b22, tT, tb2):
    B, N, _, D = g.shape
    R = MLP_ROWS
    grid = (B, N // R)
    specs = [
        pl.BlockSpec((1, R, NSAMPLE, D), lambda b, i: (b, i, 0, 0)),
        pl.BlockSpec((1, R, D), lambda b, i: (b, i, 0)),
        pl.BlockSpec((1, R, 3), lambda b, i: (b, i, 0)),
        pl.BlockSpec((3, D), lambda b, i: (0, 0)),
        pl.BlockSpec((1, D), lambda b, i: (0, 0)),
        pl.BlockSpec((D, D), lambda b, i: (0, 0)),
        pl.BlockSpec((1, D), lambda b, i: (0, 0)),
        pl.BlockSpec((D, D), lambda b, i: (0, 0)),
        pl.BlockSpec((1, D), lambda b, i: (0, 0)),
        pl.BlockSpec((D, D), lambda b, i: (0, 0)),
        pl.BlockSpec((1, D), lambda b, i: (0, 0)),
    ]
    out_spec = pl.BlockSpec((1, R, D), lambda b, i: (b, i, 0))
    return pl.pallas_call(
        _mlp_body, grid=grid, in_specs=specs, out_specs=out_spec,
        out_shape=jax.ShapeDtypeStruct((B, N, D), jnp.float32),
    )(g, featq, pcq, pos_wT, pos_b2, w1T, b12, w2T, b22, tT, tb2)


def _cross_dir(pcq, pck, featq, featk, pos_wT, pos_b2, w1T, b12, w2T, b22,
               tT, tb2):
    B, N, _ = pcq.shape
    D = featq.shape[-1]
    idx, tpre = _run_topk(pcq, pck, featk, pos_wT)
    g_flat = _sc_gather(idx.reshape(-1), tpre.reshape(B * N, D))
    g = g_flat.reshape(B, N, NSAMPLE, D)
    return _mlp_call(g, featq, pcq, pos_wT, pos_b2, w1T, b12, w2T, b22,
                     tT, tb2)


@jax.jit
def kernel(pc1, pc2, feat1, feat2, pos_w, pos_b, mlp_w1, mlp_b1,
           mlp_w2, mlp_b2, t1_w, t1_b, t2_w, t2_b):
    pos_wT = pos_w.T
    pos_b2 = pos_b.reshape(1, -1)
    w1T = mlp_w1.T
    b12 = mlp_b1.reshape(1, -1)
    w2T = mlp_w2.T
    b22 = mlp_b2.reshape(1, -1)

    f1 = _cross_dir(pc1, pc2, feat1, feat2, pos_wT, pos_b2, w1T, b12,
                    w2T, b22, t1_w.T, t1_b.reshape(1, -1))
    f2 = _cross_dir(pc2, pc1, feat2, feat1, pos_wT, pos_b2, w1T, b12,
                    w2T, b22, t2_w.T, t2_b.reshape(1, -1))
    return (f1, f2)
